# Initial kernel scaffold; baseline (speedup 1.0000x reference)
import jax
import jax.numpy as jnp
from jax.experimental import pallas as pl


def kernel(x, table):
    raise NotImplementedError("write your pallas kernel here")


# speedup vs baseline: 4.8989x; 4.8989x over previous
import functools

import jax
import jax.numpy as jnp
from jax import lax
from jax.experimental import pallas as pl
from jax.experimental.pallas import tpu as pltpu
from jax.experimental.pallas import tpu_sc as plsc

EMB = 32
SUB = 128
NSUB = 4
C = SUB * NSUB
NC, NS = 2, 16
NW = NC * NS


def _emb_kernel(n_pairs, rows_per_w, x_hbm, tab_hbm, out_hbm, idx_v, rows_v,
                gs0, gs1, os0, os1):
    wid = lax.axis_index("s") * NC + lax.axis_index("c")
    idx_row_base = wid * rows_per_w
    out_base = wid * (rows_per_w * SUB)

    def fire(b, g, gsem):
        pltpu.sync_copy(x_hbm.at[pl.ds(idx_row_base + g * NSUB, NSUB)],
                        idx_v.at[b])
        return [
            pltpu.async_copy(tab_hbm.at[idx_v.at[b, j]],
                             rows_v.at[b, pl.ds(j * SUB, SUB)], gsem)
            for j in range(NSUB)
        ]

    def write(b, g, osem):
        return pltpu.async_copy(rows_v.at[b],
                                out_hbm.at[pl.ds(out_base + g * C, C)], osem)

    def wait_writes():
        pltpu.make_async_copy(rows_v.at[0], out_hbm.at[pl.ds(0, C)],
                              os0).wait()
        pltpu.make_async_copy(rows_v.at[1], out_hbm.at[pl.ds(0, C)],
                              os1).wait()

    def body(i, _):
        pl.when(i > 0)(wait_writes)
        g0 = 2 * i
        h0 = fire(0, g0, gs0)
        h1 = fire(1, g0 + 1, gs1)
        for h in h0:
            h.wait()
        write(0, g0, os0)
        for h in h1:
            h.wait()
        write(1, g0 + 1, os1)
        return _

    lax.fori_loop(0, n_pairs, body, None)
    wait_writes()


@functools.partial(jax.jit, static_argnames=())
def _emb(xf, table):
    b_total = xf.shape[0]
    b_per_w = b_total // NW
    n_pairs = b_per_w // C // 2
    rows_per_w = b_per_w // SUB
    x2d = xf.reshape(b_total // SUB, SUB)
    mesh = plsc.VectorSubcoreMesh(core_axis_name="c", subcore_axis_name="s",
                                  num_cores=NC, num_subcores=NS)
    k = pl.kernel(
        functools.partial(_emb_kernel, n_pairs, rows_per_w),
        out_type=jax.ShapeDtypeStruct((b_total, EMB), jnp.float32),
        mesh=mesh,
        scratch_types=[
            pltpu.VMEM((2, NSUB, SUB), jnp.int32),
            pltpu.VMEM((2, C, EMB), jnp.float32),
            pltpu.SemaphoreType.DMA,
            pltpu.SemaphoreType.DMA,
            pltpu.SemaphoreType.DMA,
            pltpu.SemaphoreType.DMA,
        ],
        compiler_params=pltpu.CompilerParams(use_tc_tiling_on_sc=False),
    )
    return k(x2d, table)


def kernel(x, table):
    n, s = x.shape
    out = _emb(x.reshape(-1), table)
    return out.reshape(n, s, EMB)


# speedup vs baseline: 5.0259x; 1.0259x over previous
import functools

import jax
import jax.numpy as jnp
from jax import lax
from jax.experimental import pallas as pl
from jax.experimental.pallas import tpu as pltpu
from jax.experimental.pallas import tpu_sc as plsc

EMB = 32
SUB = 128
NSUB = 8
C = SUB * NSUB
NC, NS = 2, 16
NW = NC * NS


def _emb_kernel(n_pairs, rows_per_w, x_hbm, tab_hbm, out_hbm, idx_v, rows_v,
                gs0, gs1, os0, os1):
    wid = lax.axis_index("s") * NC + lax.axis_index("c")
    idx_row_base = wid * rows_per_w
    out_base = wid * (rows_per_w * SUB)

    def fire(b, g, gsem):
        pltpu.sync_copy(x_hbm.at[pl.ds(idx_row_base + g * NSUB, NSUB)],
                        idx_v.at[b])
        return [
            pltpu.async_copy(tab_hbm.at[idx_v.at[b, j]],
                             rows_v.at[b, pl.ds(j * SUB, SUB)], gsem)
            for j in range(NSUB)
        ]

    def write(b, g, osem):
        return pltpu.async_copy(rows_v.at[b],
                                out_hbm.at[pl.ds(out_base + g * C, C)], osem)

    def wait_writes():
        pltpu.make_async_copy(rows_v.at[0], out_hbm.at[pl.ds(0, C)],
                              os0).wait()
        pltpu.make_async_copy(rows_v.at[1], out_hbm.at[pl.ds(0, C)],
                              os1).wait()

    def body(i, _):
        pl.when(i > 0)(wait_writes)
        g0 = 2 * i
        h0 = fire(0, g0, gs0)
        h1 = fire(1, g0 + 1, gs1)
        for h in h0:
            h.wait()
        write(0, g0, os0)
        for h in h1:
            h.wait()
        write(1, g0 + 1, os1)
        return _

    lax.fori_loop(0, n_pairs, body, None)
    wait_writes()


@functools.partial(jax.jit, static_argnames=())
def _emb(xf, table):
    b_total = xf.shape[0]
    b_per_w = b_total // NW
    n_pairs = b_per_w // C // 2
    rows_per_w = b_per_w // SUB
    x2d = xf.reshape(b_total // SUB, SUB)
    mesh = plsc.VectorSubcoreMesh(core_axis_name="c", subcore_axis_name="s",
                                  num_cores=NC, num_subcores=NS)
    k = pl.kernel(
        functools.partial(_emb_kernel, n_pairs, rows_per_w),
        out_type=jax.ShapeDtypeStruct((b_total, EMB), jnp.float32),
        mesh=mesh,
        scratch_types=[
            pltpu.VMEM((2, NSUB, SUB), jnp.int32),
            pltpu.VMEM((2, C, EMB), jnp.float32),
            pltpu.SemaphoreType.DMA,
            pltpu.SemaphoreType.DMA,
            pltpu.SemaphoreType.DMA,
            pltpu.SemaphoreType.DMA,
        ],
        compiler_params=pltpu.CompilerParams(use_tc_tiling_on_sc=False),
    )
    return k(x2d, table)


def kernel(x, table):
    n, s = x.shape
    out = _emb(x.reshape(-1), table)
    return out.reshape(n, s, EMB)
